# Initial kernel scaffold; baseline (speedup 1.0000x reference)
import jax
import jax.numpy as jnp
from jax.experimental import pallas as pl


def kernel(data, hn, cn, W_ne, b_ne, W_ih, W_hh, b_ih, b_hh, W_gin, b_gin, bn1_g, bn1_b, W1, b1, bn2_g, bn2_b, prelu_a, W2, b2):
    raise NotImplementedError("write your pallas kernel here")


# speedup vs baseline: 11.4478x; 11.4478x over previous
import jax
import jax.numpy as jnp
from jax.experimental import pallas as pl
from jax.experimental.pallas import tpu as pltpu

BS = 64
N_WIN = 64
N_FEATS = 256
EMBED = 32
TOPK = 30
_F32 = jnp.float32


def _lpdot(a, b, dn):
    return jax.lax.dot_general(a, b, dn, preferred_element_type=_F32)


def _scan_step(data_ref, h0_ref, c0_ref, Wne_ref, bne_ref, Wih_ref, Whh_ref,
               bih_ref, bhh_ref, Wgin_ref, bgin_ref,
               graphs_ref, out_ref, hf_ref, cf_ref,
               h_scr, c_scr):
    b = pl.program_id(0)

    @pl.when(b == 0)
    def _():
        h_scr[...] = h0_ref[...]
        c_scr[...] = c0_ref[...]

    A = data_ref[0]
    dnT = (((1,), (0,)), ((), ()))
    noiseT = jax.nn.sigmoid(_lpdot(Wne_ref[...], A, dnT) + bne_ref[...])
    xT = A - noiseT

    hT = h_scr[...]
    cT = c_scr[...]
    zero_state = jnp.mean(hT) == 0.0
    hT_use = jnp.where(zero_state, jnp.zeros_like(hT), hT)
    cT_use = jnp.where(zero_state, jnp.zeros_like(cT), cT)

    gates = ((_lpdot(Wih_ref[...], xT, dnT) + bih_ref[...])
             + _lpdot(Whh_ref[...], hT_use, dnT)) + bhh_ref[...]
    iT = gates[0:EMBED]
    fT = gates[EMBED:2 * EMBED]
    gT = gates[2 * EMBED:3 * EMBED]
    oT = gates[3 * EMBED:4 * EMBED]
    c2 = jax.nn.sigmoid(fT) * cT_use + jax.nn.sigmoid(iT) * jnp.tanh(gT)
    h2 = jax.nn.sigmoid(oT) * jnp.tanh(c2)
    h_scr[...] = h2
    c_scr[...] = c2
    hf_ref[...] = h2
    cf_ref[...] = c2

    wT = jnp.concatenate([h2, xT], axis=0)
    w2d = jnp.transpose(wT)
    cosu = _lpdot(w2d, w2d, (((1,), (1,)), ((), ())))
    sq = w2d * w2d
    nrm_col = jnp.sqrt(jnp.sum(sq, axis=1, keepdims=True))
    nrm_row = jnp.transpose(nrm_col)
    denom = nrm_col * nrm_row
    cos = cosu / denom
    graphs_ref[0] = cos

    cosz = cos + 0.0
    bits = pltpu.bitcast(cosz, jnp.int32)
    key = jnp.where(bits >= 0, bits, bits ^ jnp.int32(0x7FFFFFFF))
    lo = jnp.min(key, axis=1, keepdims=True)
    hi = jnp.max(key, axis=1, keepdims=True) + 1

    def body(_, lohi):
        lo, hi = lohi
        T = (lo >> 1) + (hi >> 1) + (lo & hi & 1)
        cnt = jnp.sum((key >= T).astype(jnp.int32), axis=1, keepdims=True)
        ge = cnt >= TOPK
        return jnp.where(ge, T, lo), jnp.where(ge, hi, T)

    lo, hi = jax.lax.fori_loop(0, 31, body, (lo, hi))
    t30 = lo
    gt = key > t30
    eq = key == t30
    c_gt = jnp.sum(gt.astype(jnp.int32), axis=1, keepdims=True)
    r = (TOPK - c_gt).astype(_F32)
    eqf = jnp.where(eq, 1.0, 0.0)
    rowi = jax.lax.broadcasted_iota(jnp.int32, (N_FEATS, N_FEATS), 0)
    coli = jax.lax.broadcasted_iota(jnp.int32, (N_FEATS, N_FEATS), 1)
    U = jnp.where(rowi <= coli, 1.0, 0.0)
    cum = jnp.dot(eqf, U, preferred_element_type=_F32)
    M = jnp.where(gt | (eq & (cum <= r)), 1.0, 0.0)

    aggT = jnp.dot(xT, M, preferred_element_type=_F32,
                   precision=jax.lax.Precision.HIGHEST) + xT
    gin_inT = 1.1 * xT + aggT
    gcnT = _lpdot(Wgin_ref[...], gin_inT, (((1,), (0,)), ((), ()))) + bgin_ref[...]
    out_ref[...] = jnp.concatenate([gcnT, h2], axis=0)


def _tail_kernel(out_in_ref, W1_ref, b1_ref, g1_ref, bb1_ref, g2_ref, bb2_ref,
                 pa_ref, W2_ref, b2_ref, final_ref, s_out2, s_tile, s_tile2,
                 s_tile3, s_tile4):
    OUT = out_in_ref[...]
    n1 = _F32(BS * N_FEATS)
    mu1 = jnp.sum(OUT, axis=1, keepdims=True) / n1
    d1 = OUT - mu1
    var1 = jnp.sum(d1 * d1, axis=1, keepdims=True) / n1
    xn = d1 / jnp.sqrt(var1 + 1e-5) * g1_ref[...] + bb1_ref[...]
    xn = jnp.maximum(xn, 0.0)

    out2 = (jax.lax.dot_general(W1_ref[...], xn, (((1,), (0,)), ((), ())),
                                preferred_element_type=_F32)
            + b1_ref[...])
    s_out2[...] = out2

    csum = jnp.sum(out2, axis=0, keepdims=True)
    acc = jnp.zeros((1, N_FEATS), _F32)
    for bb in range(BS):
        acc = acc + csum[:, bb * N_FEATS:(bb + 1) * N_FEATS]
    n2 = _F32(BS * 256)
    mu2 = acc / n2

    vacc = jnp.zeros((1, N_FEATS), _F32)
    for bb in range(BS):
        d = s_out2[:, bb * N_FEATS:(bb + 1) * N_FEATS] - mu2
        vacc = vacc + jnp.sum(d * d, axis=0, keepdims=True)
    var2 = vacc / n2
    rstd2 = jnp.sqrt(var2 + 1e-5)

    pa = pa_ref[0, 0]
    for bb in range(BS):
        sl = slice(bb * N_FEATS, (bb + 1) * N_FEATS)
        s_tile[:, sl] = mu2
        s_tile2[:, sl] = rstd2
        s_tile3[:, sl] = g2_ref[...]
        s_tile4[:, sl] = bb2_ref[...]
    y = (s_out2[...] - s_tile[...]) / s_tile2[...] * s_tile3[...] + s_tile4[...]
    y = jnp.where(y > 0, y, pa * y)
    z = jax.lax.dot_general(W2_ref[...], y, (((1,), (0,)), ((), ())),
                            preferred_element_type=_F32) + b2_ref[0:1, 0:1]
    z = jnp.maximum(z, 0.0)
    s_tile[...] = z
    for bb in range(BS):
        final_ref[bb:bb + 1, :] = s_tile[:, bb * N_FEATS:(bb + 1) * N_FEATS]


def kernel(data, hn, cn, W_ne, b_ne, W_ih, W_hh, b_ih, b_hh, W_gin, b_gin,
           bn1_g, bn1_b, W1, b1, bn2_g, bn2_b, prelu_a, W2, b2):
    h0 = jnp.transpose(hn[0])
    c0 = jnp.transpose(cn[0])
    bne_c = b_ne[:, None]
    bih_c = b_ih[:, None]
    bhh_c = b_hh[:, None]
    bgin_c = b_gin[:, None]

    grid = (BS,)
    graphs, out_wide, hfT, cfT = pl.pallas_call(
        _scan_step,
        grid=grid,
        in_specs=[
            pl.BlockSpec((1, N_WIN, N_FEATS), lambda b: (b, 0, 0)),
            pl.BlockSpec((EMBED, N_FEATS), lambda b: (0, 0)),
            pl.BlockSpec((EMBED, N_FEATS), lambda b: (0, 0)),
            pl.BlockSpec((N_WIN, N_WIN), lambda b: (0, 0)),
            pl.BlockSpec((N_WIN, 1), lambda b: (0, 0)),
            pl.BlockSpec((4 * EMBED, N_WIN), lambda b: (0, 0)),
            pl.BlockSpec((4 * EMBED, EMBED), lambda b: (0, 0)),
            pl.BlockSpec((4 * EMBED, 1), lambda b: (0, 0)),
            pl.BlockSpec((4 * EMBED, 1), lambda b: (0, 0)),
            pl.BlockSpec((EMBED, N_WIN), lambda b: (0, 0)),
            pl.BlockSpec((EMBED, 1), lambda b: (0, 0)),
        ],
        out_specs=[
            pl.BlockSpec((1, N_FEATS, N_FEATS), lambda b: (b, 0, 0)),
            pl.BlockSpec((2 * EMBED, N_FEATS), lambda b: (0, b)),
            pl.BlockSpec((EMBED, N_FEATS), lambda b: (0, 0)),
            pl.BlockSpec((EMBED, N_FEATS), lambda b: (0, 0)),
        ],
        out_shape=[
            jax.ShapeDtypeStruct((BS, N_FEATS, N_FEATS), _F32),
            jax.ShapeDtypeStruct((2 * EMBED, BS * N_FEATS), _F32),
            jax.ShapeDtypeStruct((EMBED, N_FEATS), _F32),
            jax.ShapeDtypeStruct((EMBED, N_FEATS), _F32),
        ],
        scratch_shapes=[
            pltpu.VMEM((EMBED, N_FEATS), _F32),
            pltpu.VMEM((EMBED, N_FEATS), _F32),
        ],
        compiler_params=pltpu.CompilerParams(
            dimension_semantics=("arbitrary",)),
    )(data, h0, c0, W_ne, bne_c, W_ih, W_hh, bih_c, bhh_c, W_gin, bgin_c)

    final = pl.pallas_call(
        _tail_kernel,
        in_specs=[
            pl.BlockSpec((2 * EMBED, BS * N_FEATS), lambda: (0, 0)),
            pl.BlockSpec((256, 2 * EMBED), lambda: (0, 0)),
            pl.BlockSpec((256, 1), lambda: (0, 0)),
            pl.BlockSpec((2 * EMBED, 1), lambda: (0, 0)),
            pl.BlockSpec((2 * EMBED, 1), lambda: (0, 0)),
            pl.BlockSpec((1, N_FEATS), lambda: (0, 0)),
            pl.BlockSpec((1, N_FEATS), lambda: (0, 0)),
            pl.BlockSpec((1, 1), lambda: (0, 0)),
            pl.BlockSpec((1, 256), lambda: (0, 0)),
            pl.BlockSpec((1, 1), lambda: (0, 0)),
        ],
        out_specs=pl.BlockSpec((BS, N_FEATS), lambda: (0, 0)),
        out_shape=jax.ShapeDtypeStruct((BS, N_FEATS), _F32),
        scratch_shapes=[pltpu.VMEM((256, BS * N_FEATS), _F32),
                        pltpu.VMEM((1, BS * N_FEATS), _F32),
                        pltpu.VMEM((1, BS * N_FEATS), _F32),
                        pltpu.VMEM((1, BS * N_FEATS), _F32),
                        pltpu.VMEM((1, BS * N_FEATS), _F32)],
    )(out_wide, W1, b1[:, None], bn1_g[:, None], bn1_b[:, None],
      bn2_g[None, :], bn2_b[None, :], jnp.asarray(prelu_a).reshape(1, 1),
      W2, b2[None, :])

    hf = jax.lax.stop_gradient(jnp.transpose(hfT)[None])
    cf = jax.lax.stop_gradient(jnp.transpose(cfT)[None])
    return final, graphs, hf, cf


# speedup vs baseline: 27.7333x; 2.4226x over previous
import jax
import jax.numpy as jnp
from jax.experimental import pallas as pl
from jax.experimental.pallas import tpu as pltpu

BS = 64
N_WIN = 64
N_FEATS = 256
EMBED = 32
TOPK = 30
_F32 = jnp.float32


def _lpdot(a, b, dn):
    return jax.lax.dot_general(a, b, dn, preferred_element_type=_F32)


def _scan_step(data_ref, h0_ref, c0_ref, Wne_ref, bne_ref, Wih_ref, Whh_ref,
               bih_ref, bhh_ref, Wgin_ref, bgin_ref,
               graphs_ref, out_ref, hf_ref, cf_ref,
               h_scr, c_scr):
    b = pl.program_id(0)

    @pl.when(b == 0)
    def _():
        h_scr[...] = h0_ref[...]
        c_scr[...] = c0_ref[...]

    A = data_ref[0]
    dnT = (((1,), (0,)), ((), ()))
    noiseT = jax.nn.sigmoid(_lpdot(Wne_ref[...], A, dnT) + bne_ref[...])
    xT = A - noiseT

    hT = h_scr[...]
    cT = c_scr[...]
    zero_state = jnp.mean(hT) == 0.0
    hT_use = jnp.where(zero_state, jnp.zeros_like(hT), hT)
    cT_use = jnp.where(zero_state, jnp.zeros_like(cT), cT)

    gates = ((_lpdot(Wih_ref[...], xT, dnT) + bih_ref[...])
             + _lpdot(Whh_ref[...], hT_use, dnT)) + bhh_ref[...]
    iT = gates[0:EMBED]
    fT = gates[EMBED:2 * EMBED]
    gT = gates[2 * EMBED:3 * EMBED]
    oT = gates[3 * EMBED:4 * EMBED]
    c2 = jax.nn.sigmoid(fT) * cT_use + jax.nn.sigmoid(iT) * jnp.tanh(gT)
    h2 = jax.nn.sigmoid(oT) * jnp.tanh(c2)
    h_scr[...] = h2
    c_scr[...] = c2
    hf_ref[...] = h2
    cf_ref[...] = c2

    wT = jnp.concatenate([h2, xT], axis=0)
    w2d = jnp.transpose(wT)
    cosu = _lpdot(w2d, w2d, (((1,), (1,)), ((), ())))
    sq = w2d * w2d
    nrm_col = jnp.sqrt(jnp.sum(sq, axis=1, keepdims=True))
    nrm_row = jnp.transpose(nrm_col)
    denom = nrm_col * nrm_row
    cos = cosu / denom
    graphs_ref[0] = cos

    cosz = cos + 0.0
    bits = pltpu.bitcast(cosz, jnp.int32)
    key = jnp.where(bits >= 0, bits, bits ^ jnp.int32(0x7FFFFFFF))
    lo = jnp.min(key, axis=0, keepdims=True)
    hi = jnp.max(key, axis=0, keepdims=True) + 1

    def body(_, lohi):
        lo, hi = lohi
        T = (lo >> 1) + (hi >> 1) + (lo & hi & 1)
        cnt = jnp.sum((key >= T).astype(jnp.int32), axis=0, keepdims=True)
        ge = cnt >= TOPK
        return jnp.where(ge, T, lo), jnp.where(ge, hi, T)

    lo, hi = jax.lax.fori_loop(0, 31, body, (lo, hi))
    t30 = lo
    gt = key > t30
    eq = key == t30
    c_gt = jnp.sum(gt.astype(jnp.int32), axis=0, keepdims=True)
    r = (TOPK - c_gt).astype(_F32)
    eqf = jnp.where(eq, 1.0, 0.0)
    rowi = jax.lax.broadcasted_iota(jnp.int32, (N_FEATS, N_FEATS), 0)
    coli = jax.lax.broadcasted_iota(jnp.int32, (N_FEATS, N_FEATS), 1)
    L = jnp.where(rowi >= coli, 1.0, 0.0)
    cum = jnp.dot(L, eqf, preferred_element_type=_F32)
    MT = jnp.where(gt | (eq & (cum <= r)), 1.0, 0.0)

    aggT = jax.lax.dot_general(xT, MT, (((1,), (1,)), ((), ())),
                               preferred_element_type=_F32,
                               precision=jax.lax.Precision.HIGHEST) + xT
    gin_inT = 1.1 * xT + aggT
    gcnT = _lpdot(Wgin_ref[...], gin_inT, (((1,), (0,)), ((), ()))) + bgin_ref[...]
    out_ref[...] = jnp.concatenate([gcnT, h2], axis=0)


def _tail_kernel(out_in_ref, W1_ref, b1_ref, g1_ref, bb1_ref, g2_ref, bb2_ref,
                 pa_ref, W2_ref, b2_ref, final_ref, s_out2, s_tile, s_tile2,
                 s_tile3, s_tile4):
    OUT = out_in_ref[...]
    n1 = _F32(BS * N_FEATS)
    mu1 = jnp.sum(OUT, axis=1, keepdims=True) / n1
    d1 = OUT - mu1
    var1 = jnp.sum(d1 * d1, axis=1, keepdims=True) / n1
    xn = d1 / jnp.sqrt(var1 + 1e-5) * g1_ref[...] + bb1_ref[...]
    xn = jnp.maximum(xn, 0.0)

    out2 = (jax.lax.dot_general(W1_ref[...], xn, (((1,), (0,)), ((), ())),
                                preferred_element_type=_F32)
            + b1_ref[...])
    s_out2[...] = out2

    csum = jnp.sum(out2, axis=0, keepdims=True)
    csq = jnp.sum(out2 * out2, axis=0, keepdims=True)

    def fold(v):
        w = BS * N_FEATS // 2
        while w >= N_FEATS:
            v = v[:, :w] + v[:, w:2 * w]
            w //= 2
        return v

    n2 = _F32(BS * 256)
    mu2 = fold(csum) / n2
    var2 = fold(csq) / n2 - mu2 * mu2
    rstd2 = jnp.sqrt(var2 + 1e-5)

    pa = pa_ref[0, 0]
    for bb in range(BS):
        sl = slice(bb * N_FEATS, (bb + 1) * N_FEATS)
        s_tile[:, sl] = mu2
        s_tile2[:, sl] = rstd2
        s_tile3[:, sl] = g2_ref[...]
        s_tile4[:, sl] = bb2_ref[...]
    y = (s_out2[...] - s_tile[...]) / s_tile2[...] * s_tile3[...] + s_tile4[...]
    y = jnp.where(y > 0, y, pa * y)
    z = jax.lax.dot_general(W2_ref[...], y, (((1,), (0,)), ((), ())),
                            preferred_element_type=_F32) + b2_ref[0:1, 0:1]
    z = jnp.maximum(z, 0.0)
    s_tile[...] = z
    for bb in range(BS):
        final_ref[bb:bb + 1, :] = s_tile[:, bb * N_FEATS:(bb + 1) * N_FEATS]


def kernel(data, hn, cn, W_ne, b_ne, W_ih, W_hh, b_ih, b_hh, W_gin, b_gin,
           bn1_g, bn1_b, W1, b1, bn2_g, bn2_b, prelu_a, W2, b2):
    h0 = jnp.transpose(hn[0])
    c0 = jnp.transpose(cn[0])
    bne_c = b_ne[:, None]
    bih_c = b_ih[:, None]
    bhh_c = b_hh[:, None]
    bgin_c = b_gin[:, None]

    grid = (BS,)
    graphs, out_wide, hfT, cfT = pl.pallas_call(
        _scan_step,
        grid=grid,
        in_specs=[
            pl.BlockSpec((1, N_WIN, N_FEATS), lambda b: (b, 0, 0)),
            pl.BlockSpec((EMBED, N_FEATS), lambda b: (0, 0)),
            pl.BlockSpec((EMBED, N_FEATS), lambda b: (0, 0)),
            pl.BlockSpec((N_WIN, N_WIN), lambda b: (0, 0)),
            pl.BlockSpec((N_WIN, 1), lambda b: (0, 0)),
            pl.BlockSpec((4 * EMBED, N_WIN), lambda b: (0, 0)),
            pl.BlockSpec((4 * EMBED, EMBED), lambda b: (0, 0)),
            pl.BlockSpec((4 * EMBED, 1), lambda b: (0, 0)),
            pl.BlockSpec((4 * EMBED, 1), lambda b: (0, 0)),
            pl.BlockSpec((EMBED, N_WIN), lambda b: (0, 0)),
            pl.BlockSpec((EMBED, 1), lambda b: (0, 0)),
        ],
        out_specs=[
            pl.BlockSpec((1, N_FEATS, N_FEATS), lambda b: (b, 0, 0)),
            pl.BlockSpec((2 * EMBED, N_FEATS), lambda b: (0, b)),
            pl.BlockSpec((EMBED, N_FEATS), lambda b: (0, 0)),
            pl.BlockSpec((EMBED, N_FEATS), lambda b: (0, 0)),
        ],
        out_shape=[
            jax.ShapeDtypeStruct((BS, N_FEATS, N_FEATS), _F32),
            jax.ShapeDtypeStruct((2 * EMBED, BS * N_FEATS), _F32),
            jax.ShapeDtypeStruct((EMBED, N_FEATS), _F32),
            jax.ShapeDtypeStruct((EMBED, N_FEATS), _F32),
        ],
        scratch_shapes=[
            pltpu.VMEM((EMBED, N_FEATS), _F32),
            pltpu.VMEM((EMBED, N_FEATS), _F32),
        ],
        compiler_params=pltpu.CompilerParams(
            dimension_semantics=("arbitrary",)),
    )(data, h0, c0, W_ne, bne_c, W_ih, W_hh, bih_c, bhh_c, W_gin, bgin_c)

    final = pl.pallas_call(
        _tail_kernel,
        in_specs=[
            pl.BlockSpec((2 * EMBED, BS * N_FEATS), lambda: (0, 0)),
            pl.BlockSpec((256, 2 * EMBED), lambda: (0, 0)),
            pl.BlockSpec((256, 1), lambda: (0, 0)),
            pl.BlockSpec((2 * EMBED, 1), lambda: (0, 0)),
            pl.BlockSpec((2 * EMBED, 1), lambda: (0, 0)),
            pl.BlockSpec((1, N_FEATS), lambda: (0, 0)),
            pl.BlockSpec((1, N_FEATS), lambda: (0, 0)),
            pl.BlockSpec((1, 1), lambda: (0, 0)),
            pl.BlockSpec((1, 256), lambda: (0, 0)),
            pl.BlockSpec((1, 1), lambda: (0, 0)),
        ],
        out_specs=pl.BlockSpec((BS, N_FEATS), lambda: (0, 0)),
        out_shape=jax.ShapeDtypeStruct((BS, N_FEATS), _F32),
        scratch_shapes=[pltpu.VMEM((256, BS * N_FEATS), _F32),
                        pltpu.VMEM((1, BS * N_FEATS), _F32),
                        pltpu.VMEM((1, BS * N_FEATS), _F32),
                        pltpu.VMEM((1, BS * N_FEATS), _F32),
                        pltpu.VMEM((1, BS * N_FEATS), _F32)],
    )(out_wide, W1, b1[:, None], bn1_g[:, None], bn1_b[:, None],
      bn2_g[None, :], bn2_b[None, :], jnp.asarray(prelu_a).reshape(1, 1),
      W2, b2[None, :])

    hf = jax.lax.stop_gradient(jnp.transpose(hfT)[None])
    cf = jax.lax.stop_gradient(jnp.transpose(cfT)[None])
    return final, graphs, hf, cf


# speedup vs baseline: 28.3466x; 1.0221x over previous
import jax
import jax.numpy as jnp
from jax.experimental import pallas as pl
from jax.experimental.pallas import tpu as pltpu

BS = 64
N_WIN = 64
N_FEATS = 256
EMBED = 32
TOPK = 30
_F32 = jnp.float32


def _lpdot(a, b, dn):
    return jax.lax.dot_general(a, b, dn, preferred_element_type=_F32)


def _scan_step(data_ref, h0_ref, c0_ref, Wne_ref, bne_ref, Wih_ref, Whh_ref,
               bih_ref, bhh_ref, Wgin_ref, bgin_ref,
               graphs_ref, out_ref, hf_ref, cf_ref,
               h_scr, c_scr):
    b = pl.program_id(0)

    @pl.when(b == 0)
    def _():
        h_scr[...] = h0_ref[...]
        c_scr[...] = c0_ref[...]

    A = data_ref[0]
    dnT = (((1,), (0,)), ((), ()))
    noiseT = jax.nn.sigmoid(_lpdot(Wne_ref[...], A, dnT) + bne_ref[...])
    xT = A - noiseT

    hT = h_scr[...]
    cT = c_scr[...]
    zero_state = jnp.mean(hT) == 0.0
    hT_use = jnp.where(zero_state, jnp.zeros_like(hT), hT)
    cT_use = jnp.where(zero_state, jnp.zeros_like(cT), cT)

    gates = ((_lpdot(Wih_ref[...], xT, dnT) + bih_ref[...])
             + _lpdot(Whh_ref[...], hT_use, dnT)) + bhh_ref[...]
    iT = gates[0:EMBED]
    fT = gates[EMBED:2 * EMBED]
    gT = gates[2 * EMBED:3 * EMBED]
    oT = gates[3 * EMBED:4 * EMBED]
    c2 = jax.nn.sigmoid(fT) * cT_use + jax.nn.sigmoid(iT) * jnp.tanh(gT)
    h2 = jax.nn.sigmoid(oT) * jnp.tanh(c2)
    h_scr[...] = h2
    c_scr[...] = c2
    hf_ref[...] = h2
    cf_ref[...] = c2

    wT = jnp.concatenate([h2, xT], axis=0)
    w2d = jnp.transpose(wT)
    cosu = _lpdot(w2d, w2d, (((1,), (1,)), ((), ())))
    sq = w2d * w2d
    nrm_col = jnp.sqrt(jnp.sum(sq, axis=1, keepdims=True))
    nrm_row = jnp.transpose(nrm_col)
    denom = nrm_col * nrm_row
    cos = cosu / denom
    graphs_ref[0] = cos

    cosz = cos + 0.0
    bits = pltpu.bitcast(cosz, jnp.int32)
    key = jnp.where(bits >= 0, bits, bits ^ jnp.int32(0x7FFFFFFF))
    lo = jnp.min(key, axis=0, keepdims=True)
    hi = jnp.max(key, axis=0, keepdims=True) + 1

    for _ in range(31):
        T = (lo >> 1) + (hi >> 1) + (lo & hi & 1)
        cnt = jnp.sum((key >= T).astype(jnp.int32), axis=0, keepdims=True)
        ge = cnt >= TOPK
        lo = jnp.where(ge, T, lo)
        hi = jnp.where(ge, hi, T)
    t30 = lo
    gt = key > t30
    eq = key == t30
    c_gt = jnp.sum(gt.astype(jnp.int32), axis=0, keepdims=True)
    r = (TOPK - c_gt).astype(_F32)
    eqf = jnp.where(eq, 1.0, 0.0)
    rowi = jax.lax.broadcasted_iota(jnp.int32, (N_FEATS, N_FEATS), 0)
    coli = jax.lax.broadcasted_iota(jnp.int32, (N_FEATS, N_FEATS), 1)
    L = jnp.where(rowi >= coli, 1.0, 0.0)
    cum = jnp.dot(L, eqf, preferred_element_type=_F32)
    MT = jnp.where(gt | (eq & (cum <= r)), 1.0, 0.0)

    xh = xT.astype(jnp.bfloat16)
    xl = (xT - xh.astype(_F32)).astype(jnp.bfloat16)
    MTb = MT.astype(jnp.bfloat16)
    dnc = (((1,), (1,)), ((), ()))
    aggT = (jax.lax.dot_general(xh, MTb, dnc, preferred_element_type=_F32)
            + jax.lax.dot_general(xl, MTb, dnc, preferred_element_type=_F32)
            + xT)
    gin_inT = 1.1 * xT + aggT
    gcnT = _lpdot(Wgin_ref[...], gin_inT, (((1,), (0,)), ((), ()))) + bgin_ref[...]
    out_ref[...] = jnp.concatenate([gcnT, h2], axis=0)


def _tail_kernel(out_in_ref, W1_ref, b1_ref, g1_ref, bb1_ref, g2_ref, bb2_ref,
                 pa_ref, W2_ref, b2_ref, final_ref, s_out2, s_tile, s_tile2,
                 s_tile3, s_tile4):
    OUT = out_in_ref[...]
    n1 = _F32(BS * N_FEATS)
    mu1 = jnp.sum(OUT, axis=1, keepdims=True) / n1
    d1 = OUT - mu1
    var1 = jnp.sum(d1 * d1, axis=1, keepdims=True) / n1
    xn = d1 / jnp.sqrt(var1 + 1e-5) * g1_ref[...] + bb1_ref[...]
    xn = jnp.maximum(xn, 0.0)

    out2 = (jax.lax.dot_general(W1_ref[...], xn, (((1,), (0,)), ((), ())),
                                preferred_element_type=_F32)
            + b1_ref[...])
    s_out2[...] = out2

    csum = jnp.sum(out2, axis=0, keepdims=True)
    csq = jnp.sum(out2 * out2, axis=0, keepdims=True)

    def fold(v):
        w = BS * N_FEATS // 2
        while w >= N_FEATS:
            v = v[:, :w] + v[:, w:2 * w]
            w //= 2
        return v

    n2 = _F32(BS * 256)
    mu2 = fold(csum) / n2
    var2 = fold(csq) / n2 - mu2 * mu2
    rstd2 = jnp.sqrt(var2 + 1e-5)

    pa = pa_ref[0, 0]
    for bb in range(BS):
        sl = slice(bb * N_FEATS, (bb + 1) * N_FEATS)
        s_tile[:, sl] = mu2
        s_tile2[:, sl] = rstd2
        s_tile3[:, sl] = g2_ref[...]
        s_tile4[:, sl] = bb2_ref[...]
    y = (s_out2[...] - s_tile[...]) / s_tile2[...] * s_tile3[...] + s_tile4[...]
    y = jnp.where(y > 0, y, pa * y)
    z = jax.lax.dot_general(W2_ref[...], y, (((1,), (0,)), ((), ())),
                            preferred_element_type=_F32) + b2_ref[0:1, 0:1]
    z = jnp.maximum(z, 0.0)
    s_tile[...] = z
    for bb in range(BS):
        final_ref[bb:bb + 1, :] = s_tile[:, bb * N_FEATS:(bb + 1) * N_FEATS]


def kernel(data, hn, cn, W_ne, b_ne, W_ih, W_hh, b_ih, b_hh, W_gin, b_gin,
           bn1_g, bn1_b, W1, b1, bn2_g, bn2_b, prelu_a, W2, b2):
    h0 = jnp.transpose(hn[0])
    c0 = jnp.transpose(cn[0])
    bne_c = b_ne[:, None]
    bih_c = b_ih[:, None]
    bhh_c = b_hh[:, None]
    bgin_c = b_gin[:, None]

    grid = (BS,)
    graphs, out_wide, hfT, cfT = pl.pallas_call(
        _scan_step,
        grid=grid,
        in_specs=[
            pl.BlockSpec((1, N_WIN, N_FEATS), lambda b: (b, 0, 0)),
            pl.BlockSpec((EMBED, N_FEATS), lambda b: (0, 0)),
            pl.BlockSpec((EMBED, N_FEATS), lambda b: (0, 0)),
            pl.BlockSpec((N_WIN, N_WIN), lambda b: (0, 0)),
            pl.BlockSpec((N_WIN, 1), lambda b: (0, 0)),
            pl.BlockSpec((4 * EMBED, N_WIN), lambda b: (0, 0)),
            pl.BlockSpec((4 * EMBED, EMBED), lambda b: (0, 0)),
            pl.BlockSpec((4 * EMBED, 1), lambda b: (0, 0)),
            pl.BlockSpec((4 * EMBED, 1), lambda b: (0, 0)),
            pl.BlockSpec((EMBED, N_WIN), lambda b: (0, 0)),
            pl.BlockSpec((EMBED, 1), lambda b: (0, 0)),
        ],
        out_specs=[
            pl.BlockSpec((1, N_FEATS, N_FEATS), lambda b: (b, 0, 0)),
            pl.BlockSpec((2 * EMBED, N_FEATS), lambda b: (0, b)),
            pl.BlockSpec((EMBED, N_FEATS), lambda b: (0, 0)),
            pl.BlockSpec((EMBED, N_FEATS), lambda b: (0, 0)),
        ],
        out_shape=[
            jax.ShapeDtypeStruct((BS, N_FEATS, N_FEATS), _F32),
            jax.ShapeDtypeStruct((2 * EMBED, BS * N_FEATS), _F32),
            jax.ShapeDtypeStruct((EMBED, N_FEATS), _F32),
            jax.ShapeDtypeStruct((EMBED, N_FEATS), _F32),
        ],
        scratch_shapes=[
            pltpu.VMEM((EMBED, N_FEATS), _F32),
            pltpu.VMEM((EMBED, N_FEATS), _F32),
        ],
        compiler_params=pltpu.CompilerParams(
            dimension_semantics=("arbitrary",)),
    )(data, h0, c0, W_ne, bne_c, W_ih, W_hh, bih_c, bhh_c, W_gin, bgin_c)

    final = pl.pallas_call(
        _tail_kernel,
        in_specs=[
            pl.BlockSpec((2 * EMBED, BS * N_FEATS), lambda: (0, 0)),
            pl.BlockSpec((256, 2 * EMBED), lambda: (0, 0)),
            pl.BlockSpec((256, 1), lambda: (0, 0)),
            pl.BlockSpec((2 * EMBED, 1), lambda: (0, 0)),
            pl.BlockSpec((2 * EMBED, 1), lambda: (0, 0)),
            pl.BlockSpec((1, N_FEATS), lambda: (0, 0)),
            pl.BlockSpec((1, N_FEATS), lambda: (0, 0)),
            pl.BlockSpec((1, 1), lambda: (0, 0)),
            pl.BlockSpec((1, 256), lambda: (0, 0)),
            pl.BlockSpec((1, 1), lambda: (0, 0)),
        ],
        out_specs=pl.BlockSpec((BS, N_FEATS), lambda: (0, 0)),
        out_shape=jax.ShapeDtypeStruct((BS, N_FEATS), _F32),
        scratch_shapes=[pltpu.VMEM((256, BS * N_FEATS), _F32),
                        pltpu.VMEM((1, BS * N_FEATS), _F32),
                        pltpu.VMEM((1, BS * N_FEATS), _F32),
                        pltpu.VMEM((1, BS * N_FEATS), _F32),
                        pltpu.VMEM((1, BS * N_FEATS), _F32)],
    )(out_wide, W1, b1[:, None], bn1_g[:, None], bn1_b[:, None],
      bn2_g[None, :], bn2_b[None, :], jnp.asarray(prelu_a).reshape(1, 1),
      W2, b2[None, :])

    hf = jax.lax.stop_gradient(jnp.transpose(hfT)[None])
    cf = jax.lax.stop_gradient(jnp.transpose(cfT)[None])
    return final, graphs, hf, cf
